# baseline (speedup 1.0000x reference)
import functools

import jax
import jax.numpy as jnp
from jax import lax
from jax.experimental import pallas as pl
from jax.experimental.pallas import tpu as pltpu
from jax.experimental.pallas import tpu_sc as plsc

F32 = jnp.float32
NC = 2
NS = 16
LANE = 128


def _nrows(r_valid, r0, chunk):
    return jnp.clip(r_valid - r0, 0, chunk)


def _sc_gather(table, idx_r, e_total):
    R = idx_r.shape[0]
    r_valid = e_total // LANE
    per_w = R // (NC * NS)
    d = table.shape[1]
    mesh = plsc.VectorSubcoreMesh(core_axis_name="c", subcore_axis_name="s")

    @functools.partial(
        pl.kernel,
        mesh=mesh,
        out_type=jax.ShapeDtypeStruct((e_total, d), F32),
        scratch_types=[
            pltpu.VMEM((per_w, LANE), jnp.int32),
            pltpu.VMEM((LANE, d), F32),
            pltpu.VMEM((LANE, d), F32),
            pltpu.SemaphoreType.DMA,
            pltpu.SemaphoreType.DMA,
        ],
    )
    def k(t_h, i_h, out, idx_v, b0, b1, s0, s1):
        cid = lax.axis_index("c")
        sid = lax.axis_index("s")
        wid = cid * NS + sid
        r0 = wid * per_w
        nrows = _nrows(r_valid, r0, per_w)
        pltpu.sync_copy(i_h.at[pl.ds(r0, per_w)], idx_v)
        bufs = (b0, b1)
        sems = (s0, s1)

        def fire(j, b):
            pltpu.async_copy(t_h.at[idx_v.at[j]], bufs[b], sems[b])

        def wait(j, b):
            pltpu.make_async_copy(t_h.at[idx_v.at[j]], bufs[b],
                                  sems[b]).wait()

        fire(0, 0)

        def body(kk, carry):
            for b in range(2):
                j = 2 * kk + b

                @pl.when(j < nrows)
                def _():
                    wait(j, b)

                    @pl.when(j + 1 < nrows)
                    def _():
                        fire(j + 1, 1 - b)

                    pltpu.sync_copy(bufs[b],
                                    out.at[pl.ds((r0 + j) * LANE, LANE)])

            return carry

        lax.fori_loop(0, (per_w + 1) // 2, body, 0)

    return k(table, idx_r)


_NZ = 624


def _acc_io(copy_fn, sid):
    copy_fn(sid * _NZ, _NZ)

    @pl.when(sid == NS - 1)
    def _():
        copy_fn(NS * _NZ, 16)


def _sc_scatter_counts(vals, row_r, zeros, ones):
    R = row_r.shape[0]
    d = vals.shape[1]
    N = zeros.shape[0]
    r_valid = vals.shape[0] // LANE
    chunk = R // NS
    mesh = plsc.VectorSubcoreMesh(core_axis_name="c", subcore_axis_name="s")

    @functools.partial(
        pl.kernel,
        mesh=mesh,
        out_type=jax.ShapeDtypeStruct((N, 2 * d), F32),
        scratch_types=[
            pltpu.VMEM((chunk // 2, LANE), jnp.int32),
            pltpu.VMEM((LANE, d), F32),
            pltpu.VMEM((LANE, d), F32),
            pltpu.VMEM_SHARED((N, d), F32),
            pltpu.SemaphoreType.DMA,
            pltpu.SemaphoreType.DMA,
        ],
    )
    def k(v_h, row_h, z_h, ones_h, out_h, idx_v, v0, v1, acc, s0, s1):
        cid = lax.axis_index("c")
        sid = lax.axis_index("s")
        _acc_io(lambda o, nr: pltpu.sync_copy(
            z_h.at[pl.ds(o, nr), pl.ds(0, d)], acc.at[pl.ds(o, nr)]), sid)
        r0 = sid * chunk
        half = chunk // 2
        nrows = _nrows(r_valid, r0, chunk)
        bufs = (v0, v1)
        sems = (s0, s1)

        def src(j):
            return v_h.at[pl.ds((r0 + j) * LANE, LANE)]

        @pl.when(cid == 1)
        def _():
            pltpu.sync_copy(ones_h, v0)

        plsc.subcore_barrier()

        for h in range(2):
            base = h * half
            pend = jnp.minimum(nrows, base + half)
            pltpu.sync_copy(row_h.at[pl.ds(r0 + base, half)], idx_v)

            @pl.when(cid == 0)
            def _():
                pltpu.async_copy(src(base), bufs[0], sems[0])
                pltpu.async_copy(src(base + 1), bufs[1], sems[1])

                def body(kk, carry):
                    for b in range(2):
                        jl = 2 * kk + b
                        j = base + jl

                        @pl.when(j < pend)
                        def _():
                            pltpu.make_async_copy(src(j), bufs[b],
                                                  sems[b]).wait()
                            pltpu.sync_copy(bufs[b], acc.at[idx_v.at[jl]],
                                            add=True)

                            @pl.when(j + 2 < pend)
                            def _():
                                pltpu.async_copy(src(j + 2), bufs[b],
                                                 sems[b])

                    return carry

                lax.fori_loop(0, half // 2, body, 0)

            @pl.when(cid == 1)
            def _():
                def body(jl, carry):
                    pltpu.sync_copy(v0, acc.at[idx_v.at[jl]], add=True)
                    return carry

                lax.fori_loop(0, pend - base, body, 0)

        plsc.subcore_barrier()
        _acc_io(lambda o, nr: pltpu.sync_copy(
            acc.at[pl.ds(o, nr)], out_h.at[pl.ds(o, nr), pl.ds(cid * d, d)]),
            sid)

    return k(vals, row_r, zeros, ones)


def _sc_scatter_fsplit(vals, row_r, zeros):
    R = row_r.shape[0]
    d = vals.shape[1]
    dh = d // NC
    N = zeros.shape[0]
    r_valid = vals.shape[0] // LANE
    chunk = R // NS
    mesh = plsc.VectorSubcoreMesh(core_axis_name="c", subcore_axis_name="s")

    @functools.partial(
        pl.kernel,
        mesh=mesh,
        out_type=jax.ShapeDtypeStruct((N, d), F32),
        scratch_types=[
            pltpu.VMEM((chunk // 2, LANE), jnp.int32),
            pltpu.VMEM((LANE, dh), F32),
            pltpu.VMEM((LANE, dh), F32),
            pltpu.VMEM_SHARED((N, dh), F32),
            pltpu.SemaphoreType.DMA,
            pltpu.SemaphoreType.DMA,
        ],
    )
    def k(v_h, row_h, z_h, out_h, idx_v, v0, v1, acc, s0, s1):
        cid = lax.axis_index("c")
        sid = lax.axis_index("s")
        _acc_io(lambda o, nr: pltpu.sync_copy(
            z_h.at[pl.ds(o, nr), pl.ds(0, dh)], acc.at[pl.ds(o, nr)]), sid)
        r0 = sid * chunk
        half = chunk // 2
        nrows = _nrows(r_valid, r0, chunk)
        plsc.subcore_barrier()
        bufs = (v0, v1)
        sems = (s0, s1)

        def src(j):
            return v_h.at[pl.ds((r0 + j) * LANE, LANE), pl.ds(cid * dh, dh)]

        for h in range(2):
            base = h * half
            pend = jnp.minimum(nrows, base + half)
            pltpu.sync_copy(row_h.at[pl.ds(r0 + base, half)], idx_v)
            pltpu.async_copy(src(base), bufs[0], sems[0])
            pltpu.async_copy(src(base + 1), bufs[1], sems[1])

            def body(kk, carry):
                for b in range(2):
                    jl = 2 * kk + b
                    j = base + jl

                    @pl.when(j < pend)
                    def _():
                        pltpu.make_async_copy(src(j), bufs[b], sems[b]).wait()
                        pltpu.sync_copy(bufs[b], acc.at[idx_v.at[jl]],
                                        add=True)

                        @pl.when(j + 2 < pend)
                        def _():
                            pltpu.async_copy(src(j + 2), bufs[b], sems[b])

                return carry

            lax.fori_loop(0, half // 2, body, 0)
        plsc.subcore_barrier()
        _acc_io(lambda o, nr: pltpu.sync_copy(
            acc.at[pl.ds(o, nr)], out_h.at[pl.ds(o, nr), pl.ds(cid * dh, dh)]),
            sid)

    return k(vals, row_r, zeros)


def _full(shape):
    return pl.BlockSpec(shape, lambda i: tuple(0 for _ in shape))


def _rows(bshape):
    return pl.BlockSpec(bshape, lambda i: (i,) + tuple(0 for _ in bshape[1:]))


def _tc_node_pre(x, w1a, c1a, w1c, c1c, blk=1000):
    n = x.shape[0]

    def body(x_r, wa_r, ba_r, wc_r, bc_r, ta_r, tc_r):
        xv = x_r[...]
        ta_r[...] = jnp.dot(xv, wa_r[...], preferred_element_type=F32) + ba_r[...]
        tc_r[...] = jnp.dot(xv, wc_r[...], preferred_element_type=F32) + bc_r[...]

    return pl.pallas_call(
        body,
        grid=(n // blk,),
        in_specs=[_rows((blk, 128)), _full(w1a.shape), _full(c1a.shape),
                  _full(w1c.shape), _full(c1c.shape)],
        out_specs=[_rows((blk, w1a.shape[1])), _rows((blk, w1c.shape[1]))],
        out_shape=[jax.ShapeDtypeStruct((n, w1a.shape[1]), F32),
                   jax.ShapeDtypeStruct((n, w1c.shape[1]), F32)],
    )(x, w1a, c1a, w1c, c1c)


def _tc_edge1(a1, c1, ea, wea, w2e, b2e, w1b, w2n, b2n, blk=2000):
    e = a1.shape[0]

    def body(a_r, c_r, ea_r, wea_r, w2e_r, b2e_r, w1b_r, w2n_r, b2n_r,
             e1_r, h_r):
        bf = jnp.bfloat16
        c = c_r[...]
        u = a_r[...][:, :64] + c[:, :64] + jnp.dot(
            ea_r[...].astype(bf), wea_r[...], preferred_element_type=F32)
        e1 = jnp.dot(jnp.maximum(u, 0.0).astype(bf), w2e_r[...],
                     preferred_element_type=F32) + b2e_r[...]
        e1_r[...] = e1
        hp = c[:, 64:192] + jnp.dot(e1.astype(bf), w1b_r[...],
                                    preferred_element_type=F32)
        h_r[...] = jnp.dot(jnp.maximum(hp, 0.0).astype(bf), w2n_r[...],
                           preferred_element_type=F32) + b2n_r[...]

    return pl.pallas_call(
        body,
        grid=(e // blk,),
        in_specs=[_rows((blk, 128)), _rows((blk, 256)), _rows((blk, 16)),
                  _full(wea.shape), _full(w2e.shape), _full(b2e.shape),
                  _full(w1b.shape), _full(w2n.shape), _full(b2n.shape)],
        out_specs=[_rows((blk, 128)), _rows((blk, 128))],
        out_shape=[jax.ShapeDtypeStruct((e, 128), F32),
                   jax.ShapeDtypeStruct((e, 128), F32)],
    )(a1, c1, ea, wea, w2e, b2e, w1b, w2n, b2n)


def _tc_node1(x, s1, wn2a, wn2b, b1n2, w2n2, b2n2, a2a, ca, a2c, cc,
              a2x, cx, blk=1000):
    n = x.shape[0]

    def body(x_r, s1_r, wa_r, wb_r, b1_r, w2_r, b2_r,
             aa_r, ca_r, ac_r, cc_r, ax_r, cx_r, t2a_r, t2c_r, x2p_r):
        s1v = s1_r[...]
        cnt = s1v[:, 128:129]
        m = s1v[:, :128] / jnp.maximum(cnt, 1.0)
        pre = (jnp.dot(x_r[...], wa_r[...], preferred_element_type=F32)
               + jnp.dot(m, wb_r[...], preferred_element_type=F32) + b1_r[...])
        n1 = jnp.dot(jnp.maximum(pre, 0.0), w2_r[...],
                     preferred_element_type=F32) + b2_r[...]
        t2a_r[...] = jnp.dot(n1, aa_r[...], preferred_element_type=F32) + ca_r[...]
        t2c_r[...] = jnp.dot(n1, ac_r[...], preferred_element_type=F32) + cc_r[...]
        x2p_r[...] = jnp.dot(n1, ax_r[...], preferred_element_type=F32) + cx_r[...]

    return pl.pallas_call(
        body,
        grid=(n // blk,),
        in_specs=[_rows((blk, 128)), _rows((blk, 256)),
                  _full(wn2a.shape), _full(wn2b.shape), _full(b1n2.shape),
                  _full(w2n2.shape), _full(b2n2.shape),
                  _full(a2a.shape), _full(ca.shape), _full(a2c.shape),
                  _full(cc.shape), _full(a2x.shape), _full(cx.shape)],
        out_specs=[_rows((blk, 128)), _rows((blk, 384)), _rows((blk, 256))],
        out_shape=[jax.ShapeDtypeStruct((n, 128), F32),
                   jax.ShapeDtypeStruct((n, 384), F32),
                   jax.ShapeDtypeStruct((n, 256), F32)],
    )(x, s1, wn2a, wn2b, b1n2, w2n2, b2n2, a2a, ca, a2c, cc, a2x, cx)


def _tc_edge2(a2, c2, e1, we1, w2e, b2e, w2b, w2n, b2n, blk=2000):
    e = a2.shape[0]

    def body(a_r, c_r, e1_r, we1_r, w2e_r, b2e_r, w2b_r, w2n_r, b2n_r, h2_r):
        bf = jnp.bfloat16
        c = c_r[...]
        v = a_r[...] + c[:, :128] + jnp.dot(e1_r[...].astype(bf), we1_r[...],
                                            preferred_element_type=F32)
        e2 = jnp.dot(jnp.maximum(v, 0.0).astype(bf), w2e_r[...],
                     preferred_element_type=F32) + b2e_r[...]
        h2p = c[:, 128:] + jnp.dot(e2.astype(bf), w2b_r[...],
                                   preferred_element_type=F32)
        h2_r[...] = jnp.dot(jnp.maximum(h2p, 0.0).astype(bf), w2n_r[...],
                            preferred_element_type=F32) + b2n_r[...]

    return pl.pallas_call(
        body,
        grid=(e // blk,),
        in_specs=[_rows((blk, 128)), _rows((blk, 384)), _rows((blk, 128)),
                  _full(we1.shape), _full(w2e.shape), _full(b2e.shape),
                  _full(w2b.shape), _full(w2n.shape), _full(b2n.shape)],
        out_specs=_rows((blk, 256)),
        out_shape=jax.ShapeDtypeStruct((e, 256), F32),
    )(a2, c2, e1, we1, w2e, b2e, w2b, w2n, b2n)


def _tc_node2(x2p, s2, s1, batch3, wq, w2, b2, fc1w, fc1b, fc2w, fc2b,
              blk=1000):
    n = x2p.shape[0]
    grid = n // blk

    def body(x2p_r, s2_r, s1_r, b_r, wq_r, w2_r, b2_r,
             f1w_r, f1b_r, f2w_r, f2b_r, out_r, pooled):
        i = pl.program_id(0)
        cnt = s1_r[...][:, 128:129]
        m2 = s2_r[...] / jnp.maximum(cnt, 1.0)
        pre = x2p_r[...] + jnp.dot(m2, wq_r[...], preferred_element_type=F32)
        x2 = jnp.dot(jnp.maximum(pre, 0.0), w2_r[...],
                     preferred_element_type=F32) + b2_r[...]
        bb = b_r[0]
        oh = (bb == lax.broadcasted_iota(jnp.int32, (64, blk), 0)).astype(F32)

        @pl.when(i == 0)
        def _():
            pooled[...] = jnp.zeros_like(pooled)

        pooled[...] += jnp.dot(oh, x2, preferred_element_type=F32)

        @pl.when(i == grid - 1)
        def _():
            p = jnp.maximum(jnp.dot(pooled[...], f1w_r[...],
                                    preferred_element_type=F32) + f1b_r[...],
                            0.0)
            out_r[...] = jnp.dot(p, f2w_r[...],
                                 preferred_element_type=F32) + f2b_r[...]

    bspec = pl.BlockSpec((1, 1, blk), lambda i: (i, 0, 0))
    return pl.pallas_call(
        body,
        grid=(grid,),
        in_specs=[_rows((blk, 256)), _rows((blk, 256)), _rows((blk, 256)),
                  bspec, _full(wq.shape), _full(w2.shape), _full(b2.shape),
                  _full(fc1w.shape), _full(fc1b.shape), _full(fc2w.shape),
                  _full(fc2b.shape)],
        out_specs=_full((64, 128)),
        out_shape=jax.ShapeDtypeStruct((64, 128), F32),
        scratch_shapes=[pltpu.VMEM((64, 256), F32)],
    )(x2p, s2, s1, batch3, wq, w2, b2, fc1w, fc1b, fc2w, fc2b)


def kernel(x, edge_index, edge_attr, batch,
           l1e_w1, l1e_b1, l1e_w2, l1e_b2,
           l1n1_w1, l1n1_b1, l1n1_w2, l1n1_b2,
           l1n2_w1, l1n2_b1, l1n2_w2, l1n2_b2,
           bn_n_g, bn_n_b, bn_e_g, bn_e_b,
           l2e_w1, l2e_b1, l2e_w2, l2e_b2,
           l2n1_w1, l2n1_b1, l2n1_w2, l2n1_b2,
           l2n2_w1, l2n2_b1, l2n2_w2, l2n2_b2,
           fc1_w, fc1_b, fc2_w, fc2_b):
    n = x.shape[0]
    e = edge_attr.shape[0]
    row = edge_index[0]
    col = edge_index[1]
    r_pad = 80 * NC * NS
    pad = ((0, r_pad - e // LANE), (0, 0))
    row_r = jnp.pad(row.reshape(e // LANE, LANE), pad)
    col_r = jnp.pad(col.reshape(e // LANE, LANE), pad)
    zeros = jnp.zeros((n, 256), F32)

    inv = jnp.float32(1.0 / jnp.sqrt(1.0 + 1e-5))
    sn = inv * bn_n_g
    se = inv * bn_e_g

    w1a = jnp.pad(l1e_w1[0:128], ((0, 0), (0, 64)))
    w1c = jnp.pad(jnp.concatenate([l1e_w1[128:256], l1n1_w1[0:128]], axis=1),
                  ((0, 0), (0, 64)))
    c1a = jnp.pad(l1e_b1, (0, 64))[None, :]
    c1c = jnp.pad(jnp.concatenate([jnp.zeros((64,), F32), l1n1_b1]),
                  (0, 64))[None, :]
    wea = l1e_w1[256:272]

    wA = l2e_w1[0:256]
    wB = l2e_w1[256:512]
    wC = l2e_w1[512:640]
    wD = l2n1_w1[0:256]
    wF = l2n1_w1[256:384]
    wP = l2n2_w1[0:256]
    wQ = l2n2_w1[256:512]
    a2a = sn[:, None] * wA
    ca = (bn_n_b @ wA + bn_e_b @ wC + l2e_b1)[None, :]
    a2c = jnp.concatenate([sn[:, None] * wB, sn[:, None] * wD], axis=1)
    cc = jnp.concatenate([bn_n_b @ wB, bn_n_b @ wD + l2n1_b1])[None, :]
    a2x = sn[:, None] * wP
    cx = (bn_n_b @ wP + l2n2_b1)[None, :]
    we1 = se[:, None] * wC

    fc2w_p = jnp.pad(fc2_w, ((0, 0), (0, 127)))
    fc2b_p = jnp.pad(fc2_b, (0, 127))[None, :]

    t1a, t1c = _tc_node_pre(x, w1a, c1a, w1c, c1c)
    a1 = _sc_gather(t1a, row_r, e)
    c1 = _sc_gather(t1c, col_r, e)
    bf = jnp.bfloat16
    e1, h = _tc_edge1(a1, c1, edge_attr, wea.astype(bf), l1e_w2.astype(bf),
                      l1e_b2[None, :], l1n1_w1[128:256].astype(bf),
                      l1n1_w2.astype(bf), l1n1_b2[None, :])
    s1 = _sc_scatter_counts(h, row_r, zeros, jnp.ones((LANE, 128), F32))
    t2a, t2c, x2p = _tc_node1(x, s1, l1n2_w1[0:128], l1n2_w1[128:256],
                              l1n2_b1[None, :], l1n2_w2, l1n2_b2[None, :],
                              a2a, ca, a2c, cc, a2x, cx)
    a2 = _sc_gather(t2a, row_r, e)
    c2 = _sc_gather(t2c, col_r, e)
    h2 = _tc_edge2(a2, c2, e1, we1.astype(bf), l2e_w2.astype(bf),
                   l2e_b2[None, :], l2n1_w1[256:384].astype(bf),
                   l2n1_w2.astype(bf), l2n1_b2[None, :])
    s2 = _sc_scatter_fsplit(h2, row_r, zeros)
    batch3 = batch.reshape(n // 1000, 1, 1000)
    out = _tc_node2(x2p, s2, s1, batch3, wQ, l2n2_w2, l2n2_b2[None, :],
                    fc1_w, fc1b := fc1_b[None, :], fc2w_p, fc2b_p)
    return out[:, :1]


# speedup vs baseline: 1.1828x; 1.1828x over previous
import functools

import numpy as np

import jax
import jax.numpy as jnp
from jax import lax
from jax.experimental import pallas as pl
from jax.experimental.pallas import tpu as pltpu
from jax.experimental.pallas import tpu_sc as plsc

F32 = jnp.float32
NC = 2
NS = 16
LANE = 128


def _nrows(r_valid, r0, chunk):
    return jnp.clip(r_valid - r0, 0, chunk)


def _sc_gather(table, idx_r, e_total):
    R = idx_r.shape[0]
    r_valid = e_total // LANE
    per_w = R // (NC * NS)
    tail = table.shape[1:]
    dt = table.dtype
    mesh = plsc.VectorSubcoreMesh(core_axis_name="c", subcore_axis_name="s")

    @functools.partial(
        pl.kernel,
        mesh=mesh,
        out_type=jax.ShapeDtypeStruct((e_total,) + tail, dt),
        scratch_types=[
            pltpu.VMEM((per_w, LANE), jnp.int32),
            pltpu.VMEM((LANE,) + tail, dt),
            pltpu.VMEM((LANE,) + tail, dt),
            pltpu.SemaphoreType.DMA,
            pltpu.SemaphoreType.DMA,
        ],
    )
    def k(t_h, i_h, out, idx_v, b0, b1, s0, s1):
        cid = lax.axis_index("c")
        sid = lax.axis_index("s")
        wid = cid * NS + sid
        r0 = wid * per_w
        nrows = _nrows(r_valid, r0, per_w)
        pltpu.sync_copy(i_h.at[pl.ds(r0, per_w)], idx_v)
        bufs = (b0, b1)
        sems = (s0, s1)

        def fire(j, b):
            pltpu.async_copy(t_h.at[idx_v.at[j]], bufs[b], sems[b])

        def wait(j, b):
            pltpu.make_async_copy(t_h.at[idx_v.at[j]], bufs[b],
                                  sems[b]).wait()

        fire(0, 0)

        def body(kk, carry):
            for b in range(2):
                j = 2 * kk + b

                @pl.when(j < nrows)
                def _():
                    wait(j, b)

                    @pl.when(j + 1 < nrows)
                    def _():
                        fire(j + 1, 1 - b)

                    pltpu.sync_copy(bufs[b],
                                    out.at[pl.ds((r0 + j) * LANE, LANE)])

            return carry

        lax.fori_loop(0, (per_w + 1) // 2, body, 0)

    return k(table, idx_r)


_NZ = 624


def _acc_io(copy_fn, sid):
    copy_fn(sid * _NZ, _NZ)

    @pl.when(sid == NS - 1)
    def _():
        copy_fn(NS * _NZ, 16)


def _sc_scatter_counts(vals, row_r, zeros, ones):
    R = row_r.shape[0]
    d = vals.shape[1]
    N = zeros.shape[0]
    r_valid = vals.shape[0] // LANE
    chunk = R // NS
    mesh = plsc.VectorSubcoreMesh(core_axis_name="c", subcore_axis_name="s")

    @functools.partial(
        pl.kernel,
        mesh=mesh,
        out_type=jax.ShapeDtypeStruct((N, 2 * d), F32),
        scratch_types=[
            pltpu.VMEM((chunk // 2, LANE), jnp.int32),
            pltpu.VMEM((LANE, d), F32),
            pltpu.VMEM((LANE, d), F32),
            pltpu.VMEM_SHARED((N, d), F32),
            pltpu.SemaphoreType.DMA,
            pltpu.SemaphoreType.DMA,
        ],
    )
    def k(v_h, row_h, z_h, ones_h, out_h, idx_v, v0, v1, acc, s0, s1):
        cid = lax.axis_index("c")
        sid = lax.axis_index("s")
        _acc_io(lambda o, nr: pltpu.sync_copy(
            z_h.at[pl.ds(o, nr), pl.ds(0, d)], acc.at[pl.ds(o, nr)]), sid)
        r0 = sid * chunk
        half = chunk // 2
        nrows = _nrows(r_valid, r0, chunk)
        bufs = (v0, v1)
        sems = (s0, s1)

        def src(j):
            return v_h.at[pl.ds((r0 + j) * LANE, LANE)]

        @pl.when(cid == 1)
        def _():
            pltpu.sync_copy(ones_h, v0)

        plsc.subcore_barrier()

        for h in range(2):
            base = h * half
            pend = jnp.minimum(nrows, base + half)
            pltpu.sync_copy(row_h.at[pl.ds(r0 + base, half)], idx_v)

            @pl.when(cid == 0)
            def _():
                pltpu.async_copy(src(base), bufs[0], sems[0])
                pltpu.async_copy(src(base + 1), bufs[1], sems[1])

                def body(kk, carry):
                    for b in range(2):
                        jl = 2 * kk + b
                        j = base + jl

                        @pl.when(j < pend)
                        def _():
                            pltpu.make_async_copy(src(j), bufs[b],
                                                  sems[b]).wait()
                            pltpu.sync_copy(bufs[b], acc.at[idx_v.at[jl]],
                                            add=True)

                            @pl.when(j + 2 < pend)
                            def _():
                                pltpu.async_copy(src(j + 2), bufs[b],
                                                 sems[b])

                    return carry

                lax.fori_loop(0, half // 2, body, 0)

            @pl.when(cid == 1)
            def _():
                def body(jl, carry):
                    pltpu.sync_copy(v0, acc.at[idx_v.at[jl]], add=True)
                    return carry

                lax.fori_loop(0, pend - base, body, 0)

        plsc.subcore_barrier()
        _acc_io(lambda o, nr: pltpu.sync_copy(
            acc.at[pl.ds(o, nr)], out_h.at[pl.ds(o, nr), pl.ds(cid * d, d)]),
            sid)

    return k(vals, row_r, zeros, ones)


def _sc_scatter_fsplit(vals, row_r, zeros):
    R = row_r.shape[0]
    d = vals.shape[1]
    dh = d // NC
    N = zeros.shape[0]
    r_valid = vals.shape[0] // LANE
    chunk = R // NS
    mesh = plsc.VectorSubcoreMesh(core_axis_name="c", subcore_axis_name="s")

    @functools.partial(
        pl.kernel,
        mesh=mesh,
        out_type=jax.ShapeDtypeStruct((N, d), F32),
        scratch_types=[
            pltpu.VMEM((chunk // 2, LANE), jnp.int32),
            pltpu.VMEM((LANE, dh), F32),
            pltpu.VMEM((LANE, dh), F32),
            pltpu.VMEM_SHARED((N, dh), F32),
            pltpu.SemaphoreType.DMA,
            pltpu.SemaphoreType.DMA,
        ],
    )
    def k(v_h, row_h, z_h, out_h, idx_v, v0, v1, acc, s0, s1):
        cid = lax.axis_index("c")
        sid = lax.axis_index("s")
        _acc_io(lambda o, nr: pltpu.sync_copy(
            z_h.at[pl.ds(o, nr), pl.ds(0, dh)], acc.at[pl.ds(o, nr)]), sid)
        r0 = sid * chunk
        half = chunk // 2
        nrows = _nrows(r_valid, r0, chunk)
        plsc.subcore_barrier()
        bufs = (v0, v1)
        sems = (s0, s1)

        def src(j):
            return v_h.at[pl.ds((r0 + j) * LANE, LANE), pl.ds(cid * dh, dh)]

        for h in range(2):
            base = h * half
            pend = jnp.minimum(nrows, base + half)
            pltpu.sync_copy(row_h.at[pl.ds(r0 + base, half)], idx_v)
            pltpu.async_copy(src(base), bufs[0], sems[0])
            pltpu.async_copy(src(base + 1), bufs[1], sems[1])

            def body(kk, carry):
                for b in range(2):
                    jl = 2 * kk + b
                    j = base + jl

                    @pl.when(j < pend)
                    def _():
                        pltpu.make_async_copy(src(j), bufs[b], sems[b]).wait()
                        pltpu.sync_copy(bufs[b], acc.at[idx_v.at[jl]],
                                        add=True)

                        @pl.when(j + 2 < pend)
                        def _():
                            pltpu.async_copy(src(j + 2), bufs[b], sems[b])

                return carry

            lax.fori_loop(0, half // 2, body, 0)
        plsc.subcore_barrier()
        _acc_io(lambda o, nr: pltpu.sync_copy(
            acc.at[pl.ds(o, nr)], out_h.at[pl.ds(o, nr), pl.ds(cid * dh, dh)]),
            sid)

    return k(vals, row_r, zeros)


def _pack2(x0, x1):
    u0 = jax.lax.bitcast_convert_type(x0, jnp.uint32)
    u1 = jax.lax.bitcast_convert_type(x1, jnp.uint32)
    r0 = (u0 + 0x7FFF + ((u0 >> 16) & 1)) >> 16
    r1 = (u1 + 0x7FFF + ((u1 >> 16) & 1)) & np.uint32(0xFFFF0000)
    return jax.lax.bitcast_convert_type(r0 | r1, jnp.int32)


def _unpack2(p):
    u = jax.lax.bitcast_convert_type(p, jnp.uint32)
    x0 = jax.lax.bitcast_convert_type(u << 16, F32)
    x1 = jax.lax.bitcast_convert_type(u & np.uint32(0xFFFF0000), F32)
    return x0, x1


def _full(shape):
    return pl.BlockSpec(shape, lambda i: tuple(0 for _ in shape))


def _rows(bshape):
    return pl.BlockSpec(bshape, lambda i: (i,) + tuple(0 for _ in bshape[1:]))


def _tc_node_pre(x, w1a, c1a, w1c, c1c, blk=2000):
    n = x.shape[0]

    def body(x_r, wa_r, ba_r, wc_r, bc_r, ta_r, tc_r):
        xv = x_r[...]
        ta_r[...] = jnp.dot(xv, wa_r[...], preferred_element_type=F32) + ba_r[...]
        v = jnp.dot(xv, wc_r[...], preferred_element_type=F32) + bc_r[...]
        tc_r[...] = _pack2(v[:, :128], v[:, 128:256])

    return pl.pallas_call(
        body,
        grid=(n // blk,),
        in_specs=[_rows((blk, 128)), _full(w1a.shape), _full(c1a.shape),
                  _full(w1c.shape), _full(c1c.shape)],
        out_specs=[_rows((blk, w1a.shape[1])), _rows((blk, 128))],
        out_shape=[jax.ShapeDtypeStruct((n, w1a.shape[1]), F32),
                   jax.ShapeDtypeStruct((n, 128), jnp.int32)],
    )(x, w1a, c1a, w1c, c1c)


def _tc_edge1(a1, c1, ea, wea, w2e, b2e, w1b, w2n, b2n, blk=2000):
    e = a1.shape[0]

    def body(a_r, c_r, ea_r, wea_r, w2e_r, b2e_r, w1b_r, w2n_r, b2n_r,
             e1_r, h_r):
        c0, c1v = _unpack2(c_r[...])
        c = jnp.concatenate([c0, c1v], axis=1)
        u = a_r[...][:, :64] + c[:, :64] + jnp.dot(ea_r[...], wea_r[...],
                                                   preferred_element_type=F32)
        e1 = jnp.dot(jnp.maximum(u, 0.0), w2e_r[...],
                     preferred_element_type=F32) + b2e_r[...]
        e1_r[...] = e1.astype(jnp.bfloat16)
        hp = c[:, 64:192] + jnp.dot(e1, w1b_r[...], preferred_element_type=F32)
        h_r[...] = jnp.dot(jnp.maximum(hp, 0.0), w2n_r[...],
                           preferred_element_type=F32) + b2n_r[...]

    return pl.pallas_call(
        body,
        grid=(e // blk,),
        in_specs=[_rows((blk, 128)), _rows((blk, 128)), _rows((blk, 16)),
                  _full(wea.shape), _full(w2e.shape), _full(b2e.shape),
                  _full(w1b.shape), _full(w2n.shape), _full(b2n.shape)],
        out_specs=[_rows((blk, 128)), _rows((blk, 128))],
        out_shape=[jax.ShapeDtypeStruct((e, 128), jnp.bfloat16),
                   jax.ShapeDtypeStruct((e, 128), F32)],
    )(a1, c1, ea, wea, w2e, b2e, w1b, w2n, b2n)


def _tc_node1(x, s1, wn2a, wn2b, b1n2, w2n2, b2n2, a2a, ca, a2c, cc,
              a2x, cx, blk=2000):
    n = x.shape[0]

    def body(x_r, s1_r, wa_r, wb_r, b1_r, w2_r, b2_r,
             aa_r, ca_r, ac_r, cc_r, ax_r, cx_r, t2a_r, t2c_r, x2p_r):
        s1v = s1_r[...]
        cnt = s1v[:, 128:129]
        m = s1v[:, :128] / jnp.maximum(cnt, 1.0)
        pre = (jnp.dot(x_r[...], wa_r[...], preferred_element_type=F32)
               + jnp.dot(m, wb_r[...], preferred_element_type=F32) + b1_r[...])
        n1 = jnp.dot(jnp.maximum(pre, 0.0), w2_r[...],
                     preferred_element_type=F32) + b2_r[...]
        t2a_r[...] = jnp.dot(n1, aa_r[...], preferred_element_type=F32) + ca_r[...]
        v = jnp.dot(n1, ac_r[...], preferred_element_type=F32) + cc_r[...]
        pk = _pack2(v[:, :192], v[:, 192:384])
        t2c_r[...] = jnp.concatenate(
            [pk, jnp.zeros((pk.shape[0], 64), jnp.int32)], axis=1)
        x2p_r[...] = jnp.dot(n1, ax_r[...], preferred_element_type=F32) + cx_r[...]

    return pl.pallas_call(
        body,
        grid=(n // blk,),
        in_specs=[_rows((blk, 128)), _rows((blk, 256)),
                  _full(wn2a.shape), _full(wn2b.shape), _full(b1n2.shape),
                  _full(w2n2.shape), _full(b2n2.shape),
                  _full(a2a.shape), _full(ca.shape), _full(a2c.shape),
                  _full(cc.shape), _full(a2x.shape), _full(cx.shape)],
        out_specs=[_rows((blk, 128)), _rows((blk, 256)), _rows((blk, 256))],
        out_shape=[jax.ShapeDtypeStruct((n, 128), F32),
                   jax.ShapeDtypeStruct((n, 256), jnp.int32),
                   jax.ShapeDtypeStruct((n, 256), F32)],
    )(x, s1, wn2a, wn2b, b1n2, w2n2, b2n2, a2a, ca, a2c, cc, a2x, cx)


def _tc_edge2(a2, c2, e1, we1, w2e, b2e, w2b, w2n, b2n, blk=2000):
    e = a2.shape[0]

    def body(a_r, c_r, e1_r, we1_r, w2e_r, b2e_r, w2b_r, w2n_r, b2n_r, h2_r):
        c0, c1v = _unpack2(c_r[...][:, :192])
        c = jnp.concatenate([c0, c1v], axis=1)
        v = a_r[...] + c[:, :128] + jnp.dot(e1_r[...], we1_r[...],
                                            preferred_element_type=F32)
        e2 = jnp.dot(jnp.maximum(v, 0.0), w2e_r[...],
                     preferred_element_type=F32) + b2e_r[...]
        h2p = c[:, 128:] + jnp.dot(e2, w2b_r[...], preferred_element_type=F32)
        h2_r[...] = jnp.dot(jnp.maximum(h2p, 0.0), w2n_r[...],
                            preferred_element_type=F32) + b2n_r[...]

    return pl.pallas_call(
        body,
        grid=(e // blk,),
        in_specs=[_rows((blk, 128)), _rows((blk, 256)), _rows((blk, 128)),
                  _full(we1.shape), _full(w2e.shape), _full(b2e.shape),
                  _full(w2b.shape), _full(w2n.shape), _full(b2n.shape)],
        out_specs=_rows((blk, 256)),
        out_shape=jax.ShapeDtypeStruct((e, 256), F32),
    )(a2, c2, e1, we1, w2e, b2e, w2b, w2n, b2n)


def _tc_node2(x2p, s2, s1, batch3, wq, w2, b2, fc1w, fc1b, fc2w, fc2b,
              blk=1000):
    n = x2p.shape[0]
    grid = n // blk

    def body(x2p_r, s2_r, s1_r, b_r, wq_r, w2_r, b2_r,
             f1w_r, f1b_r, f2w_r, f2b_r, out_r, pooled):
        i = pl.program_id(0)
        cnt = s1_r[...][:, 128:129]
        m2 = s2_r[...] / jnp.maximum(cnt, 1.0)
        pre = x2p_r[...] + jnp.dot(m2, wq_r[...], preferred_element_type=F32)
        x2 = jnp.dot(jnp.maximum(pre, 0.0), w2_r[...],
                     preferred_element_type=F32) + b2_r[...]
        bb = b_r[0]
        oh = (bb == lax.broadcasted_iota(jnp.int32, (64, blk), 0)).astype(F32)

        @pl.when(i == 0)
        def _():
            pooled[...] = jnp.zeros_like(pooled)

        pooled[...] += jnp.dot(oh, x2, preferred_element_type=F32)

        @pl.when(i == grid - 1)
        def _():
            p = jnp.maximum(jnp.dot(pooled[...], f1w_r[...],
                                    preferred_element_type=F32) + f1b_r[...],
                            0.0)
            out_r[...] = jnp.dot(p, f2w_r[...],
                                 preferred_element_type=F32) + f2b_r[...]

    bspec = pl.BlockSpec((1, 1, blk), lambda i: (i, 0, 0))
    return pl.pallas_call(
        body,
        grid=(grid,),
        in_specs=[_rows((blk, 256)), _rows((blk, 256)), _rows((blk, 256)),
                  bspec, _full(wq.shape), _full(w2.shape), _full(b2.shape),
                  _full(fc1w.shape), _full(fc1b.shape), _full(fc2w.shape),
                  _full(fc2b.shape)],
        out_specs=_full((64, 128)),
        out_shape=jax.ShapeDtypeStruct((64, 128), F32),
        scratch_shapes=[pltpu.VMEM((64, 256), F32)],
    )(x2p, s2, s1, batch3, wq, w2, b2, fc1w, fc1b, fc2w, fc2b)


def kernel(x, edge_index, edge_attr, batch,
           l1e_w1, l1e_b1, l1e_w2, l1e_b2,
           l1n1_w1, l1n1_b1, l1n1_w2, l1n1_b2,
           l1n2_w1, l1n2_b1, l1n2_w2, l1n2_b2,
           bn_n_g, bn_n_b, bn_e_g, bn_e_b,
           l2e_w1, l2e_b1, l2e_w2, l2e_b2,
           l2n1_w1, l2n1_b1, l2n1_w2, l2n1_b2,
           l2n2_w1, l2n2_b1, l2n2_w2, l2n2_b2,
           fc1_w, fc1_b, fc2_w, fc2_b):
    n = x.shape[0]
    e = edge_attr.shape[0]
    row = edge_index[0]
    col = edge_index[1]
    r_pad = 80 * NC * NS
    pad = ((0, r_pad - e // LANE), (0, 0))
    row_r = jnp.pad(row.reshape(e // LANE, LANE), pad)
    col_r = jnp.pad(col.reshape(e // LANE, LANE), pad)
    zeros = jnp.zeros((n, 256), F32)

    inv = jnp.float32(1.0 / jnp.sqrt(1.0 + 1e-5))
    sn = inv * bn_n_g
    se = inv * bn_e_g

    w1a = jnp.pad(l1e_w1[0:128], ((0, 0), (0, 64)))
    w1c = jnp.pad(jnp.concatenate([l1e_w1[128:256], l1n1_w1[0:128]], axis=1),
                  ((0, 0), (0, 64)))
    c1a = jnp.pad(l1e_b1, (0, 64))[None, :]
    c1c = jnp.pad(jnp.concatenate([jnp.zeros((64,), F32), l1n1_b1]),
                  (0, 64))[None, :]
    wea = l1e_w1[256:272]

    wA = l2e_w1[0:256]
    wB = l2e_w1[256:512]
    wC = l2e_w1[512:640]
    wD = l2n1_w1[0:256]
    wF = l2n1_w1[256:384]
    wP = l2n2_w1[0:256]
    wQ = l2n2_w1[256:512]
    a2a = sn[:, None] * wA
    ca = (bn_n_b @ wA + bn_e_b @ wC + l2e_b1)[None, :]
    a2c = jnp.concatenate([sn[:, None] * wB, sn[:, None] * wD], axis=1)
    cc = jnp.concatenate([bn_n_b @ wB, bn_n_b @ wD + l2n1_b1])[None, :]
    a2x = sn[:, None] * wP
    cx = (bn_n_b @ wP + l2n2_b1)[None, :]
    we1 = se[:, None] * wC

    fc2w_p = jnp.pad(fc2_w, ((0, 0), (0, 127)))
    fc2b_p = jnp.pad(fc2_b, (0, 127))[None, :]

    t1a, t1c = _tc_node_pre(x, w1a, c1a, w1c, c1c)
    a1 = _sc_gather(t1a, row_r, e)
    c1 = _sc_gather(t1c, col_r, e)
    bf = jnp.bfloat16
    e1, h = _tc_edge1(a1, c1, edge_attr, wea, l1e_w2,
                      l1e_b2[None, :], l1n1_w1[128:256],
                      l1n1_w2, l1n1_b2[None, :])
    s1 = _sc_scatter_counts(h, row_r, zeros, jnp.ones((LANE, 128), F32))
    t2a, t2c, x2p = _tc_node1(x, s1, l1n2_w1[0:128], l1n2_w1[128:256],
                              l1n2_b1[None, :], l1n2_w2, l1n2_b2[None, :],
                              a2a, ca, a2c, cc, a2x, cx)
    a2 = _sc_gather(t2a, row_r, e)
    c2 = _sc_gather(t2c, col_r, e)
    h2 = _tc_edge2(a2, c2, e1, we1.astype(bf), l2e_w2,
                   l2e_b2[None, :], l2n1_w1[256:384],
                   l2n1_w2, l2n1_b2[None, :])
    s2 = _sc_scatter_fsplit(h2, row_r, zeros)
    batch3 = batch.reshape(n // 1000, 1, 1000)
    out = _tc_node2(x2p, s2, s1, batch3, wQ, l2n2_w2, l2n2_b2[None, :],
                    fc1_w, fc1b := fc1_b[None, :], fc2w_p, fc2b_p)
    return out[:, :1]
